# baseline (speedup 1.0000x reference)
import functools

import jax
import jax.numpy as jnp
from jax.experimental import pallas as pl

HIDDEN = 512
K = 64
J = HIDDEN // K
BLK = 512


def _dot(a, b):
    return jax.lax.dot_general(
        a, b, (((1,), (0,)), ((), ())),
        preferred_element_type=jnp.float32,
        precision=jax.lax.Precision.HIGHEST,
    )


def _vqvae_block(x_ref, w1t_ref, b1_ref, w2t_ref, b2_ref, wbig_ref,
                 wbigt_ref, embw_ref, w3t_ref, b3_ref, w4t_ref, b4_ref,
                 recon_ref, ze_ref, emb_ref):
    h1 = jnp.maximum(_dot(x_ref[...], w1t_ref[...]) + b1_ref[...], 0.0)
    ze = _dot(h1, w2t_ref[...]) + b2_ref[...]
    ze_ref[...] = ze

    dots = _dot(ze, wbig_ref[...])
    w2n = jnp.sum(embw_ref[...] * embw_ref[...], axis=0, keepdims=True)
    w2t = jnp.concatenate([w2n] * J, axis=1)
    scores = w2t - 2.0 * dots

    parts = []
    for j in range(J):
        sj = scores[:, j * K:(j + 1) * K]
        m = jnp.min(sj, axis=-1, keepdims=True)
        kio = jax.lax.broadcasted_iota(jnp.int32, sj.shape, 1)
        cand = jnp.where(sj <= m, kio, K)
        idx = jnp.min(cand, axis=-1, keepdims=True)
        parts.append((kio == idx).astype(jnp.float32))
    onehot = jnp.concatenate(parts, axis=1)

    q = _dot(onehot, wbigt_ref[...])
    t = (q + ze) - ze
    emb_ref[...] = t

    h3 = jnp.maximum(_dot(t, w3t_ref[...]) + b3_ref[...], 0.0)
    recon_ref[...] = jnp.tanh(_dot(h3, w4t_ref[...]) + b4_ref[...])


@functools.partial(jax.jit, static_argnames=())
def kernel(x, W1, b1, W2, b2, W3, b3, W4, b4, emb_w):
    B, F = x.shape
    D = emb_w.shape[0]
    w1t, w2t_, w3t, w4t = W1.T, W2.T, W3.T, W4.T
    eyej = jnp.eye(J, dtype=emb_w.dtype)
    wbig = (eyej[None, :, :, None] * emb_w[:, None, None, :]).reshape(
        D * J, J * K)
    wbigt = wbig.T
    b1r, b2r, b3r, b4r = (b[None, :] for b in (b1, b2, b3, b4))

    n_blocks = B // BLK
    full = lambda s: pl.BlockSpec(s, lambda i: (0, 0))
    grid_spec = pl.GridSpec(
        grid=(n_blocks,),
        in_specs=[
            pl.BlockSpec((BLK, F), lambda i: (i, 0)),
            full(w1t.shape), full(b1r.shape),
            full(w2t_.shape), full(b2r.shape),
            full(wbig.shape), full(wbigt.shape), full(emb_w.shape),
            full(w3t.shape), full(b3r.shape),
            full(w4t.shape), full(b4r.shape),
        ],
        out_specs=[
            pl.BlockSpec((BLK, F), lambda i: (i, 0)),
            pl.BlockSpec((BLK, HIDDEN), lambda i: (i, 0)),
            pl.BlockSpec((BLK, HIDDEN), lambda i: (i, 0)),
        ],
    )
    recon, ze, emb = pl.pallas_call(
        _vqvae_block,
        grid_spec=grid_spec,
        out_shape=[
            jax.ShapeDtypeStruct((B, F), jnp.float32),
            jax.ShapeDtypeStruct((B, HIDDEN), jnp.float32),
            jax.ShapeDtypeStruct((B, HIDDEN), jnp.float32),
        ],
    )(x, w1t, b1r, w2t_, b2r, wbig, wbigt, emb_w, w3t, b3r, w4t, b4r)
    return recon, ze.reshape(B, D, J), emb


# speedup vs baseline: 1.9750x; 1.9750x over previous
import functools

import jax
import jax.numpy as jnp
from jax.experimental import pallas as pl

HIDDEN = 512
K = 64
J = HIDDEN // K
BLK = 512


def _dot(a, b):
    return jax.lax.dot_general(
        a, b, (((1,), (0,)), ((), ())),
        preferred_element_type=jnp.float32,
        precision=jax.lax.Precision.DEFAULT,
    )


def _vqvae_block(x_ref, w1t_ref, b1_ref, w2t_ref, b2_ref, wbig_ref,
                 wbigt_ref, embw_ref, w3t_ref, b3_ref, w4t_ref, b4_ref,
                 recon_ref, ze_ref, emb_ref):
    h1 = jnp.maximum(_dot(x_ref[...], w1t_ref[...]) + b1_ref[...], 0.0)
    ze = _dot(h1, w2t_ref[...]) + b2_ref[...]
    ze_ref[...] = ze

    dots = _dot(ze, wbig_ref[...])
    w2n = jnp.sum(embw_ref[...] * embw_ref[...], axis=0, keepdims=True)
    w2t = jnp.concatenate([w2n] * J, axis=1)
    scores = w2t - 2.0 * dots

    parts = []
    for j in range(J):
        sj = scores[:, j * K:(j + 1) * K]
        m = jnp.min(sj, axis=-1, keepdims=True)
        kio = jax.lax.broadcasted_iota(jnp.int32, sj.shape, 1)
        cand = jnp.where(sj <= m, kio, K)
        idx = jnp.min(cand, axis=-1, keepdims=True)
        parts.append((kio == idx).astype(jnp.float32))
    onehot = jnp.concatenate(parts, axis=1)

    q = _dot(onehot, wbigt_ref[...])
    t = (q + ze) - ze
    emb_ref[...] = t

    h3 = jnp.maximum(_dot(t, w3t_ref[...]) + b3_ref[...], 0.0)
    recon_ref[...] = jnp.tanh(_dot(h3, w4t_ref[...]) + b4_ref[...])


@functools.partial(jax.jit, static_argnames=())
def kernel(x, W1, b1, W2, b2, W3, b3, W4, b4, emb_w):
    B, F = x.shape
    D = emb_w.shape[0]
    w1t, w2t_, w3t, w4t = W1.T, W2.T, W3.T, W4.T
    eyej = jnp.eye(J, dtype=emb_w.dtype)
    wbig = (eyej[None, :, :, None] * emb_w[:, None, None, :]).reshape(
        D * J, J * K)
    wbigt = wbig.T
    b1r, b2r, b3r, b4r = (b[None, :] for b in (b1, b2, b3, b4))

    n_blocks = B // BLK
    full = lambda s: pl.BlockSpec(s, lambda i: (0, 0))
    grid_spec = pl.GridSpec(
        grid=(n_blocks,),
        in_specs=[
            pl.BlockSpec((BLK, F), lambda i: (i, 0)),
            full(w1t.shape), full(b1r.shape),
            full(w2t_.shape), full(b2r.shape),
            full(wbig.shape), full(wbigt.shape), full(emb_w.shape),
            full(w3t.shape), full(b3r.shape),
            full(w4t.shape), full(b4r.shape),
        ],
        out_specs=[
            pl.BlockSpec((BLK, F), lambda i: (i, 0)),
            pl.BlockSpec((BLK, HIDDEN), lambda i: (i, 0)),
            pl.BlockSpec((BLK, HIDDEN), lambda i: (i, 0)),
        ],
    )
    recon, ze, emb = pl.pallas_call(
        _vqvae_block,
        grid_spec=grid_spec,
        out_shape=[
            jax.ShapeDtypeStruct((B, F), jnp.float32),
            jax.ShapeDtypeStruct((B, HIDDEN), jnp.float32),
            jax.ShapeDtypeStruct((B, HIDDEN), jnp.float32),
        ],
    )(x, w1t, b1r, w2t_, b2r, wbig, wbigt, emb_w, w3t, b3r, w4t, b4r)
    return recon, ze.reshape(B, D, J), emb


# speedup vs baseline: 1.9786x; 1.0018x over previous
import functools

import jax
import jax.numpy as jnp
from jax.experimental import pallas as pl
from jax.experimental.pallas import tpu as pltpu

HIDDEN = 512
K = 64
J = HIDDEN // K
BLK = 512


def _dot(a, b):
    return jax.lax.dot_general(
        a, b, (((1,), (0,)), ((), ())),
        preferred_element_type=jnp.float32,
        precision=jax.lax.Precision.DEFAULT,
    )


def _vqvae_block(x_ref, w1t_ref, b1_ref, w2t_ref, b2_ref, wbig_ref,
                 wbigt_ref, embw_ref, w3t_ref, b3_ref, w4t_ref, b4_ref,
                 recon_ref, ze_ref, emb_ref):
    h1 = jnp.maximum(_dot(x_ref[...], w1t_ref[...]) + b1_ref[...], 0.0)
    ze = _dot(h1, w2t_ref[...]) + b2_ref[...]
    ze_ref[...] = ze

    dots = _dot(ze, wbig_ref[...])
    w2n = jnp.sum(embw_ref[...] * embw_ref[...], axis=0, keepdims=True)
    w2t = jnp.concatenate([w2n] * J, axis=1)
    scores = w2t - 2.0 * dots

    parts = []
    for j in range(J):
        sj = scores[:, j * K:(j + 1) * K]
        m = jnp.min(sj, axis=-1, keepdims=True)
        kio = jax.lax.broadcasted_iota(jnp.int32, sj.shape, 1)
        cand = jnp.where(sj <= m, kio, K)
        idx = jnp.min(cand, axis=-1, keepdims=True)
        parts.append((kio == idx).astype(jnp.float32))
    onehot = jnp.concatenate(parts, axis=1)

    q = _dot(onehot, wbigt_ref[...])
    t = (q + ze) - ze
    emb_ref[...] = t

    h3 = jnp.maximum(_dot(t, w3t_ref[...]) + b3_ref[...], 0.0)
    recon_ref[...] = jnp.tanh(_dot(h3, w4t_ref[...]) + b4_ref[...])


@functools.partial(jax.jit, static_argnames=())
def kernel(x, W1, b1, W2, b2, W3, b3, W4, b4, emb_w):
    B, F = x.shape
    D = emb_w.shape[0]
    w1t, w2t_, w3t, w4t = W1.T, W2.T, W3.T, W4.T
    eyej = jnp.eye(J, dtype=emb_w.dtype)
    wbig = (eyej[None, :, :, None] * emb_w[:, None, None, :]).reshape(
        D * J, J * K)
    wbigt = wbig.T
    b1r, b2r, b3r, b4r = (b[None, :] for b in (b1, b2, b3, b4))

    n_blocks = B // BLK
    full = lambda s: pl.BlockSpec(s, lambda i: (0, 0))
    grid_spec = pl.GridSpec(
        grid=(n_blocks,),
        in_specs=[
            pl.BlockSpec((BLK, F), lambda i: (i, 0)),
            full(w1t.shape), full(b1r.shape),
            full(w2t_.shape), full(b2r.shape),
            full(wbig.shape), full(wbigt.shape), full(emb_w.shape),
            full(w3t.shape), full(b3r.shape),
            full(w4t.shape), full(b4r.shape),
        ],
        out_specs=[
            pl.BlockSpec((BLK, F), lambda i: (i, 0)),
            pl.BlockSpec((BLK, HIDDEN), lambda i: (i, 0)),
            pl.BlockSpec((BLK, HIDDEN), lambda i: (i, 0)),
        ],
    )
    recon, ze, emb = pl.pallas_call(
        _vqvae_block,
        grid_spec=grid_spec,
        compiler_params=pltpu.CompilerParams(
            dimension_semantics=("parallel",),
        ),
        out_shape=[
            jax.ShapeDtypeStruct((B, F), jnp.float32),
            jax.ShapeDtypeStruct((B, HIDDEN), jnp.float32),
            jax.ShapeDtypeStruct((B, HIDDEN), jnp.float32),
        ],
    )(x, w1t, b1r, w2t_, b2r, wbig, wbigt, emb_w, w3t, b3r, w4t, b4r)
    return recon, ze.reshape(B, D, J), emb


# speedup vs baseline: 5.7573x; 2.9098x over previous
import functools

import jax
import jax.numpy as jnp
from jax.experimental import pallas as pl
from jax.experimental.pallas import tpu as pltpu

HIDDEN = 512
K = 64
J = HIDDEN // K
BLK = 512


def _dot(a, b):
    return jax.lax.dot_general(
        a, b, (((1,), (0,)), ((), ())),
        preferred_element_type=jnp.float32,
        precision=jax.lax.Precision.DEFAULT,
    )


def _vqvae_block(xt_ref, w1_ref, b1_ref, w2_ref, b2_ref, wbig_ref,
                 wbigt_ref, embwt_ref, w3_ref, b3_ref, w4_ref, b4_ref,
                 recont_ref, zet_ref, embt_ref):
    h1 = jnp.maximum(_dot(w1_ref[...], xt_ref[...]) + b1_ref[...], 0.0)
    ze = _dot(w2_ref[...], h1) + b2_ref[...]
    zet_ref[...] = ze.reshape(K, J, ze.shape[1])

    dots = _dot(wbigt_ref[...], ze)
    w2n = jnp.sum(embwt_ref[...] * embwt_ref[...], axis=1, keepdims=True)
    w2tile = jnp.concatenate([w2n] * J, axis=0)
    scores = w2tile - 2.0 * dots

    parts = []
    for j in range(J):
        sj = scores[j * K:(j + 1) * K, :]
        m = jnp.min(sj, axis=0, keepdims=True)
        kio = jax.lax.broadcasted_iota(jnp.int32, sj.shape, 0)
        cand = jnp.where(sj <= m, kio, K)
        idx = jnp.min(cand, axis=0, keepdims=True)
        parts.append((kio == idx).astype(jnp.float32))
    onehot = jnp.concatenate(parts, axis=0)

    q = _dot(wbig_ref[...], onehot)
    t = (q + ze) - ze
    embt_ref[...] = t.T

    h3 = jnp.maximum(_dot(w3_ref[...], t) + b3_ref[...], 0.0)
    recont_ref[...] = jnp.tanh(_dot(w4_ref[...], h3) + b4_ref[...])


@functools.partial(jax.jit, static_argnames=())
def kernel(x, W1, b1, W2, b2, W3, b3, W4, b4, emb_w):
    B, F = x.shape
    D = emb_w.shape[0]
    xt = x.T
    eyej = jnp.eye(J, dtype=emb_w.dtype)
    wbig = (eyej[None, :, :, None] * emb_w[:, None, None, :]).reshape(
        D * J, J * K)
    wbigt = wbig.T
    embwt = emb_w.T
    b1c, b2c, b3c, b4c = (b[:, None] for b in (b1, b2, b3, b4))

    n_blocks = B // BLK
    full = lambda s: pl.BlockSpec(s, lambda i: tuple(0 for _ in s))
    grid_spec = pl.GridSpec(
        grid=(n_blocks,),
        in_specs=[
            pl.BlockSpec((F, BLK), lambda i: (0, i)),
            full(W1.shape), full(b1c.shape),
            full(W2.shape), full(b2c.shape),
            full(wbig.shape), full(wbigt.shape), full(embwt.shape),
            full(W3.shape), full(b3c.shape),
            full(W4.shape), full(b4c.shape),
        ],
        out_specs=[
            pl.BlockSpec((F, BLK), lambda i: (0, i)),
            pl.BlockSpec((K, J, BLK), lambda i: (0, 0, i)),
            pl.BlockSpec((BLK, HIDDEN), lambda i: (i, 0)),
        ],
    )
    recont, zet, embt = pl.pallas_call(
        _vqvae_block,
        grid_spec=grid_spec,
        compiler_params=pltpu.CompilerParams(
            dimension_semantics=("parallel",),
        ),
        out_shape=[
            jax.ShapeDtypeStruct((F, B), jnp.float32),
            jax.ShapeDtypeStruct((D, J, B), jnp.float32),
            jax.ShapeDtypeStruct((B, HIDDEN), jnp.float32),
        ],
    )(xt, W1, b1c, W2, b2c, wbig, wbigt, embwt, W3, b3c, W4, b4c)
    return recont.T, jnp.transpose(zet, (2, 0, 1)), embt


# speedup vs baseline: 6.9929x; 1.2146x over previous
import functools

import jax
import jax.numpy as jnp
from jax.experimental import pallas as pl
from jax.experimental.pallas import tpu as pltpu

HIDDEN = 512
K = 64
J = HIDDEN // K
BLK = 1024


def _dot(a, b):
    return jax.lax.dot_general(
        a, b, (((1,), (0,)), ((), ())),
        preferred_element_type=jnp.float32,
        precision=jax.lax.Precision.DEFAULT,
    )


def _vqvae_block(xt_ref, w1_ref, b1_ref, w2_ref, b2_ref, wbig_ref,
                 wbigt_ref, embwt_ref, w3_ref, b3_ref, w4_ref, b4_ref,
                 recont_ref, zet_ref, embt_ref):
    h1 = jnp.maximum(_dot(w1_ref[...], xt_ref[...]) + b1_ref[...], 0.0)
    ze = _dot(w2_ref[...], h1) + b2_ref[...]
    zet_ref[...] = ze.reshape(K, J, ze.shape[1])

    dots = _dot(wbigt_ref[...], ze)
    w2n = jnp.sum(embwt_ref[...] * embwt_ref[...], axis=1, keepdims=True)
    w2tile = jnp.concatenate([w2n] * J, axis=0)
    scores = w2tile - 2.0 * dots

    parts = []
    for j in range(J):
        sj = scores[j * K:(j + 1) * K, :]
        m = jnp.min(sj, axis=0, keepdims=True)
        kio = jax.lax.broadcasted_iota(jnp.int32, sj.shape, 0)
        cand = jnp.where(sj <= m, kio, K)
        idx = jnp.min(cand, axis=0, keepdims=True)
        parts.append((kio == idx).astype(jnp.float32))
    onehot = jnp.concatenate(parts, axis=0)

    q = _dot(wbig_ref[...], onehot)
    t = (q + ze) - ze
    embt_ref[...] = t.T

    h3 = jnp.maximum(_dot(w3_ref[...], t) + b3_ref[...], 0.0)
    recont_ref[...] = jnp.tanh(_dot(w4_ref[...], h3) + b4_ref[...])


@functools.partial(jax.jit, static_argnames=())
def kernel(x, W1, b1, W2, b2, W3, b3, W4, b4, emb_w):
    B, F = x.shape
    D = emb_w.shape[0]
    xt = x.T
    eyej = jnp.eye(J, dtype=emb_w.dtype)
    wbig = (eyej[None, :, :, None] * emb_w[:, None, None, :]).reshape(
        D * J, J * K)
    wbigt = wbig.T
    embwt = emb_w.T
    b1c, b2c, b3c, b4c = (b[:, None] for b in (b1, b2, b3, b4))

    n_blocks = B // BLK
    full = lambda s: pl.BlockSpec(s, lambda i: tuple(0 for _ in s))
    grid_spec = pl.GridSpec(
        grid=(n_blocks,),
        in_specs=[
            pl.BlockSpec((F, BLK), lambda i: (0, i)),
            full(W1.shape), full(b1c.shape),
            full(W2.shape), full(b2c.shape),
            full(wbig.shape), full(wbigt.shape), full(embwt.shape),
            full(W3.shape), full(b3c.shape),
            full(W4.shape), full(b4c.shape),
        ],
        out_specs=[
            pl.BlockSpec((F, BLK), lambda i: (0, i)),
            pl.BlockSpec((K, J, BLK), lambda i: (0, 0, i)),
            pl.BlockSpec((BLK, HIDDEN), lambda i: (i, 0)),
        ],
    )
    recont, zet, embt = pl.pallas_call(
        _vqvae_block,
        grid_spec=grid_spec,
        compiler_params=pltpu.CompilerParams(
            dimension_semantics=("parallel",),
        ),
        out_shape=[
            jax.ShapeDtypeStruct((F, B), jnp.float32),
            jax.ShapeDtypeStruct((D, J, B), jnp.float32),
            jax.ShapeDtypeStruct((B, HIDDEN), jnp.float32),
        ],
    )(xt, W1, b1c, W2, b2c, wbig, wbigt, embwt, W3, b3c, W4, b4c)
    return recont.T, jnp.transpose(zet, (2, 0, 1)), embt


# speedup vs baseline: 7.1208x; 1.0183x over previous
import functools

import jax
import jax.numpy as jnp
from jax.experimental import pallas as pl
from jax.experimental.pallas import tpu as pltpu

HIDDEN = 512
K = 64
J = HIDDEN // K
BLK = 2048


def _dot(a, b):
    return jax.lax.dot_general(
        a, b, (((1,), (0,)), ((), ())),
        preferred_element_type=jnp.float32,
        precision=jax.lax.Precision.DEFAULT,
    )


def _vqvae_block(xt_ref, w1_ref, b1_ref, w2_ref, b2_ref, wbig_ref,
                 wbigt_ref, embwt_ref, w3_ref, b3_ref, w4_ref, b4_ref,
                 recont_ref, zet_ref, embt_ref):
    h1 = jnp.maximum(_dot(w1_ref[...], xt_ref[...]) + b1_ref[...], 0.0)
    ze = _dot(w2_ref[...], h1) + b2_ref[...]
    zet_ref[...] = ze.reshape(K, J, ze.shape[1])

    dots = _dot(wbigt_ref[...], ze)
    w2n = jnp.sum(embwt_ref[...] * embwt_ref[...], axis=1, keepdims=True)
    w2tile = jnp.concatenate([w2n] * J, axis=0)
    scores = w2tile - 2.0 * dots

    parts = []
    for j in range(J):
        sj = scores[j * K:(j + 1) * K, :]
        m = jnp.min(sj, axis=0, keepdims=True)
        kio = jax.lax.broadcasted_iota(jnp.int32, sj.shape, 0)
        cand = jnp.where(sj <= m, kio, K)
        idx = jnp.min(cand, axis=0, keepdims=True)
        parts.append((kio == idx).astype(jnp.float32))
    onehot = jnp.concatenate(parts, axis=0)

    q = _dot(wbig_ref[...], onehot)
    t = (q + ze) - ze
    embt_ref[...] = t.T

    h3 = jnp.maximum(_dot(w3_ref[...], t) + b3_ref[...], 0.0)
    recont_ref[...] = jnp.tanh(_dot(w4_ref[...], h3) + b4_ref[...])


@functools.partial(jax.jit, static_argnames=())
def kernel(x, W1, b1, W2, b2, W3, b3, W4, b4, emb_w):
    B, F = x.shape
    D = emb_w.shape[0]
    xt = x.T
    eyej = jnp.eye(J, dtype=emb_w.dtype)
    wbig = (eyej[None, :, :, None] * emb_w[:, None, None, :]).reshape(
        D * J, J * K)
    wbigt = wbig.T
    embwt = emb_w.T
    b1c, b2c, b3c, b4c = (b[:, None] for b in (b1, b2, b3, b4))

    n_blocks = B // BLK
    full = lambda s: pl.BlockSpec(s, lambda i: tuple(0 for _ in s))
    grid_spec = pl.GridSpec(
        grid=(n_blocks,),
        in_specs=[
            pl.BlockSpec((F, BLK), lambda i: (0, i)),
            full(W1.shape), full(b1c.shape),
            full(W2.shape), full(b2c.shape),
            full(wbig.shape), full(wbigt.shape), full(embwt.shape),
            full(W3.shape), full(b3c.shape),
            full(W4.shape), full(b4c.shape),
        ],
        out_specs=[
            pl.BlockSpec((F, BLK), lambda i: (0, i)),
            pl.BlockSpec((K, J, BLK), lambda i: (0, 0, i)),
            pl.BlockSpec((BLK, HIDDEN), lambda i: (i, 0)),
        ],
    )
    recont, zet, embt = pl.pallas_call(
        _vqvae_block,
        grid_spec=grid_spec,
        compiler_params=pltpu.CompilerParams(
            dimension_semantics=("parallel",),
        ),
        out_shape=[
            jax.ShapeDtypeStruct((F, B), jnp.float32),
            jax.ShapeDtypeStruct((D, J, B), jnp.float32),
            jax.ShapeDtypeStruct((B, HIDDEN), jnp.float32),
        ],
    )(xt, W1, b1c, W2, b2c, wbig, wbigt, embwt, W3, b3c, W4, b4c)
    return recont.T, jnp.transpose(zet, (2, 0, 1)), embt


# speedup vs baseline: 7.4146x; 1.0413x over previous
import functools

import jax
import jax.numpy as jnp
from jax.experimental import pallas as pl
from jax.experimental.pallas import tpu as pltpu

HIDDEN = 512
K = 64
J = HIDDEN // K
BLK = 2048


def _dot(a, b):
    return jax.lax.dot_general(
        a, b, (((1,), (0,)), ((), ())),
        preferred_element_type=jnp.float32,
        precision=jax.lax.Precision.DEFAULT,
    )


def _dot_tl(at, b):
    return jax.lax.dot_general(
        at, b, (((0,), (0,)), ((), ())),
        preferred_element_type=jnp.float32,
        precision=jax.lax.Precision.DEFAULT,
    )


def _vqvae_block(xt_ref, w1_ref, b1_ref, w2t_ref, b2_ref, wbig_ref,
                 wbigt_ref, embwt_ref, w3_ref, b3_ref, w4t_ref, b4_ref,
                 recont_ref, zet_ref, embt_ref):
    h1 = jnp.maximum(_dot(w1_ref[...], xt_ref[...]) + b1_ref[...], 0.0)
    ze = _dot_tl(w2t_ref[...], h1) + b2_ref[...]
    zet_ref[...] = ze.reshape(K, J, ze.shape[1])

    dots = _dot(wbigt_ref[...], ze)
    w2n = jnp.sum(embwt_ref[...] * embwt_ref[...], axis=1, keepdims=True)
    w2tile = jnp.concatenate([w2n] * J, axis=0)
    scores = w2tile - 2.0 * dots

    parts = []
    for j in range(J):
        sj = scores[j * K:(j + 1) * K, :]
        m = jnp.min(sj, axis=0, keepdims=True)
        kio = jax.lax.broadcasted_iota(jnp.int32, sj.shape, 0)
        cand = jnp.where(sj <= m, kio, K)
        idx = jnp.min(cand, axis=0, keepdims=True)
        parts.append((kio == idx).astype(jnp.float32))
    onehot = jnp.concatenate(parts, axis=0)

    q = _dot(wbig_ref[...], onehot)
    t = (q + ze) - ze
    embt_ref[...] = t.T

    h3 = jnp.maximum(_dot(w3_ref[...], t) + b3_ref[...], 0.0)
    recont_ref[...] = jnp.tanh(_dot_tl(w4t_ref[...], h3) + b4_ref[...])


@functools.partial(jax.jit, static_argnames=())
def kernel(x, W1, b1, W2, b2, W3, b3, W4, b4, emb_w):
    B, F = x.shape
    D = emb_w.shape[0]
    xt = x.T
    eyej = jnp.eye(J, dtype=emb_w.dtype)
    wbig = (eyej[None, :, :, None] * emb_w[:, None, None, :]).reshape(
        D * J, J * K)
    wbigt = wbig.T
    embwt = emb_w.T
    w2t = W2.T
    w4t = W4.T
    b1c, b2c, b3c, b4c = (b[:, None] for b in (b1, b2, b3, b4))

    n_blocks = B // BLK
    full = lambda s: pl.BlockSpec(s, lambda i: tuple(0 for _ in s))
    grid_spec = pl.GridSpec(
        grid=(n_blocks,),
        in_specs=[
            pl.BlockSpec((F, BLK), lambda i: (0, i)),
            full(W1.shape), full(b1c.shape),
            full(w2t.shape), full(b2c.shape),
            full(wbig.shape), full(wbigt.shape), full(embwt.shape),
            full(W3.shape), full(b3c.shape),
            full(w4t.shape), full(b4c.shape),
        ],
        out_specs=[
            pl.BlockSpec((F, BLK), lambda i: (0, i)),
            pl.BlockSpec((K, J, BLK), lambda i: (0, 0, i)),
            pl.BlockSpec((BLK, HIDDEN), lambda i: (i, 0)),
        ],
    )
    recont, zet, embt = pl.pallas_call(
        _vqvae_block,
        grid_spec=grid_spec,
        compiler_params=pltpu.CompilerParams(
            dimension_semantics=("parallel",),
        ),
        out_shape=[
            jax.ShapeDtypeStruct((F, B), jnp.float32),
            jax.ShapeDtypeStruct((D, J, B), jnp.float32),
            jax.ShapeDtypeStruct((B, HIDDEN), jnp.float32),
        ],
    )(xt, W1, b1c, w2t, b2c, wbig, wbigt, embwt, W3, b3c, w4t, b4c)
    return recont.T, jnp.transpose(zet, (2, 0, 1)), embt


# speedup vs baseline: 7.8890x; 1.0640x over previous
import functools

import jax
import jax.numpy as jnp
from jax.experimental import pallas as pl
from jax.experimental.pallas import tpu as pltpu

HIDDEN = 512
K = 64
J = HIDDEN // K
BLK = 2048


def _dot(a, b):
    return jax.lax.dot_general(
        a, b, (((1,), (0,)), ((), ())),
        preferred_element_type=jnp.float32,
        precision=jax.lax.Precision.DEFAULT,
    )


def _dot_tl(at, b):
    return jax.lax.dot_general(
        at, b, (((0,), (0,)), ((), ())),
        preferred_element_type=jnp.float32,
        precision=jax.lax.Precision.DEFAULT,
    )


def _vqvae_block(xt_ref, w1_ref, w2t_ref, wbig_ref,
                 wbigt_ref, w3_ref, w4t_ref, ball_ref,
                 recont_ref, zet_ref, embt_ref):
    b1c = ball_ref[0:400, :]
    b2c = ball_ref[400:912, :]
    b3c = ball_ref[912:1312, :]
    b4c = ball_ref[1312:2096, :]
    h1 = jnp.maximum(_dot(w1_ref[...], xt_ref[...]) + b1c, 0.0)
    ze = _dot_tl(w2t_ref[...], h1) + b2c
    zet_ref[...] = ze.reshape(K, J, ze.shape[1])

    dots = _dot(wbigt_ref[...], ze)
    wb = wbigt_ref[...]
    w2tile = jnp.sum(wb * wb, axis=1, keepdims=True)
    scores = w2tile - 2.0 * dots

    parts = []
    for j in range(J):
        sj = scores[j * K:(j + 1) * K, :]
        m = jnp.min(sj, axis=0, keepdims=True)
        kio = jax.lax.broadcasted_iota(jnp.int32, sj.shape, 0)
        cand = jnp.where(sj <= m, kio, K)
        idx = jnp.min(cand, axis=0, keepdims=True)
        parts.append((kio == idx).astype(jnp.float32))
    onehot = jnp.concatenate(parts, axis=0)

    q = _dot(wbig_ref[...], onehot)
    t = (q + ze) - ze
    embt_ref[...] = t.T

    h3 = jnp.maximum(_dot(w3_ref[...], t) + b3c, 0.0)
    recont_ref[...] = jnp.tanh(_dot_tl(w4t_ref[...], h3) + b4c)


@functools.partial(jax.jit, static_argnames=())
def kernel(x, W1, b1, W2, b2, W3, b3, W4, b4, emb_w):
    B, F = x.shape
    D = emb_w.shape[0]
    xt = x.T
    eyej = jnp.eye(J, dtype=emb_w.dtype)
    wbig = (eyej[None, :, :, None] * emb_w[:, None, None, :]).reshape(
        D * J, J * K)
    wbigt = wbig.T
    w2t = W2.T
    w4t = W4.T
    ball = jnp.concatenate([b1, b2, b3, b4])[:, None]

    n_blocks = B // BLK
    full = lambda s: pl.BlockSpec(s, lambda i: tuple(0 for _ in s))
    grid_spec = pl.GridSpec(
        grid=(n_blocks,),
        in_specs=[
            pl.BlockSpec((F, BLK), lambda i: (0, i)),
            full(W1.shape), full(w2t.shape),
            full(wbig.shape), full(wbigt.shape),
            full(W3.shape), full(w4t.shape), full(ball.shape),
        ],
        out_specs=[
            pl.BlockSpec((F, BLK), lambda i: (0, i)),
            pl.BlockSpec((K, J, BLK), lambda i: (0, 0, i)),
            pl.BlockSpec((BLK, HIDDEN), lambda i: (i, 0)),
        ],
    )
    recont, zet, embt = pl.pallas_call(
        _vqvae_block,
        grid_spec=grid_spec,
        compiler_params=pltpu.CompilerParams(
            dimension_semantics=("parallel",),
        ),
        out_shape=[
            jax.ShapeDtypeStruct((F, B), jnp.float32),
            jax.ShapeDtypeStruct((D, J, B), jnp.float32),
            jax.ShapeDtypeStruct((B, HIDDEN), jnp.float32),
        ],
    )(xt, W1, w2t, wbig, wbigt, W3, w4t, ball)
    return recont.T, jnp.transpose(zet, (2, 0, 1)), embt


# speedup vs baseline: 8.2382x; 1.0443x over previous
import functools

import jax
import jax.numpy as jnp
from jax.experimental import pallas as pl
from jax.experimental.pallas import tpu as pltpu

HIDDEN = 512
K = 64
J = HIDDEN // K
BLK = 2048


def _dot(a, b):
    return jax.lax.dot_general(
        a, b, (((1,), (0,)), ((), ())),
        preferred_element_type=jnp.float32,
        precision=jax.lax.Precision.DEFAULT,
    )


def _dot_tl(at, b):
    return jax.lax.dot_general(
        at, b, (((0,), (0,)), ((), ())),
        preferred_element_type=jnp.float32,
        precision=jax.lax.Precision.DEFAULT,
    )


def _vqvae_block(xt_ref, w1_ref, w2t_ref, embw_ref, w3_ref, w4t_ref,
                 ball_ref, recont_ref, zet_ref, embt_ref, wbigt_ref):
    @pl.when(pl.program_id(0) == 0)
    def _build():
        ewt = embw_ref[...].T
        e8 = jnp.concatenate([ewt] * J, axis=1)
        et = jnp.concatenate([e8] * J, axis=0)
        rows = jax.lax.broadcasted_iota(jnp.int32, et.shape, 0)
        cols = jax.lax.broadcasted_iota(jnp.int32, et.shape, 1)
        masked = jnp.where((rows >> 6) == (cols >> 6), et, 0.0)
        perm = (((rows & 63) == (cols >> 3)) &
                ((rows >> 6) == (cols & 7))).astype(jnp.float32)
        wbigt_ref[...] = _dot(masked, perm)

    b1c = ball_ref[0:400, :]
    b2c = ball_ref[400:912, :]
    b3c = ball_ref[912:1312, :]
    b4c = ball_ref[1312:2096, :]
    h1 = jnp.maximum(_dot(w1_ref[...], xt_ref[...]) + b1c, 0.0)
    ze = _dot_tl(w2t_ref[...], h1) + b2c
    zet_ref[...] = ze.reshape(K, J, ze.shape[1])

    wb = wbigt_ref[...]
    dots = _dot(wb, ze)
    w2tile = jnp.sum(wb * wb, axis=1, keepdims=True)
    scores = w2tile - 2.0 * dots

    parts = []
    for j in range(J):
        sj = scores[j * K:(j + 1) * K, :]
        m = jnp.min(sj, axis=0, keepdims=True)
        kio = jax.lax.broadcasted_iota(jnp.int32, sj.shape, 0)
        cand = jnp.where(sj <= m, kio, K)
        idx = jnp.min(cand, axis=0, keepdims=True)
        parts.append((kio == idx).astype(jnp.float32))
    onehot = jnp.concatenate(parts, axis=0)

    q = _dot_tl(wb, onehot)
    t = (q + ze) - ze
    embt_ref[...] = t.T

    h3 = jnp.maximum(_dot(w3_ref[...], t) + b3c, 0.0)
    recont_ref[...] = jnp.tanh(_dot_tl(w4t_ref[...], h3) + b4c)


@functools.partial(jax.jit, static_argnames=())
def kernel(x, W1, b1, W2, b2, W3, b3, W4, b4, emb_w):
    B, F = x.shape
    D = emb_w.shape[0]
    xt = x.T
    w2t = W2.T
    w4t = W4.T
    ball = jnp.concatenate([b1, b2, b3, b4])[:, None]

    n_blocks = B // BLK
    full = lambda s: pl.BlockSpec(s, lambda i: tuple(0 for _ in s))
    recont, zet, embt = pl.pallas_call(
        _vqvae_block,
        grid=(n_blocks,),
        in_specs=[
            pl.BlockSpec((F, BLK), lambda i: (0, i)),
            full(W1.shape), full(w2t.shape), full(emb_w.shape),
            full(W3.shape), full(w4t.shape), full(ball.shape),
        ],
        out_specs=[
            pl.BlockSpec((F, BLK), lambda i: (0, i)),
            pl.BlockSpec((K, J, BLK), lambda i: (0, 0, i)),
            pl.BlockSpec((BLK, HIDDEN), lambda i: (i, 0)),
        ],
        compiler_params=pltpu.CompilerParams(
            dimension_semantics=("arbitrary",),
        ),
        out_shape=[
            jax.ShapeDtypeStruct((F, B), jnp.float32),
            jax.ShapeDtypeStruct((D, J, B), jnp.float32),
            jax.ShapeDtypeStruct((B, HIDDEN), jnp.float32),
        ],
        scratch_shapes=[pltpu.VMEM((J * K, D * J), jnp.float32)],
    )(xt, W1, w2t, emb_w, W3, w4t, ball)
    return recont.T, jnp.transpose(zet, (2, 0, 1)), embt
